# Initial kernel scaffold; baseline (speedup 1.0000x reference)
import jax
import jax.numpy as jnp
from jax.experimental import pallas as pl


def kernel(x, edge_index, W, att_W):
    raise NotImplementedError("write your pallas kernel here")


# speedup vs baseline: 11.9494x; 11.9494x over previous
import functools

import jax
import jax.numpy as jnp
from jax import lax
from jax.experimental import pallas as pl
from jax.experimental.pallas import tpu as pltpu
from jax.experimental.pallas import tpu_sc as plsc

N = 10000
E = 160000
F = 256
FH = 128
NC = 2
NS = 16
L = 16
EPT = E // NS
CH = 128
NFULL = EPT // CH
REM = EPT - NFULL * CH
NBLK = N // CH
NREM = N - NBLK * CH
MAXB = (NBLK + NS - 1) // NS


def _tc_body(x_ref, w_ref, att_ref, wx2_ref, a_ref):
    xb = x_ref[...]
    wx = lax.dot_general(xb, w_ref[...], (((1,), (1,)), ((), ())),
                         preferred_element_type=jnp.float32)
    wx2_ref[0] = wx[:, :FH]
    wx2_ref[1] = wx[:, FH:]
    ab = jnp.dot(wx, att_ref[...], preferred_element_type=jnp.float32)
    a_ref[0] = ab[:, :L]
    a_ref[1] = ab[:, L:]


def _tc_matmul(x, W, att16):
    nb = 1000
    grid = N // nb
    return pl.pallas_call(
        _tc_body,
        grid=(grid,),
        in_specs=[
            pl.BlockSpec((nb, F), lambda i: (i, 0)),
            pl.BlockSpec((F, F), lambda i: (0, 0)),
            pl.BlockSpec((F, 2 * L), lambda i: (0, 0)),
        ],
        out_specs=[
            pl.BlockSpec((NC, nb, FH), lambda i: (0, i, 0)),
            pl.BlockSpec((2, nb, L), lambda i: (0, i, 0)),
        ],
        out_shape=[
            jax.ShapeDtypeStruct((NC, N, FH), jnp.float32),
            jax.ShapeDtypeStruct((2, N, L), jnp.float32),
        ],
    )(x, W, att16)


def _sc_body(wx_ref, a_ref, src_ref, dst_ref, out_ref,
             sidx, didx, wxidx, dnidx, sidx16, didx16, wxidx16, dnidx16,
             avs, avd, expc, dv, rows, coeff, qbuf,
             denom_sp, acc_sp):
    outbuf = rows
    c = lax.axis_index("c")
    s = lax.axis_index("s")
    iota = lax.iota(jnp.int32, L)
    head_mask = iota < 4
    zero16 = jnp.zeros((L,), jnp.float32)
    ebase = pl.multiple_of(s * EPT, EPT)

    def zero_rows(k, carry):
        for l in range(FH // L):
            rows[k, pl.ds(l * L, L)] = zero16
        expc[k, :] = zero16
        return carry
    lax.fori_loop(0, CH, zero_rows, 0)

    def z_blk(k, carry):
        b = s + k * NS

        @pl.when(b < NBLK)
        def _():
            nb0 = pl.multiple_of(b * CH, CH)
            pltpu.sync_copy(rows, acc_sp.at[pl.ds(nb0, CH), :])
            pltpu.sync_copy(expc, denom_sp.at[pl.ds(nb0, CH), :])
        return carry
    lax.fori_loop(0, MAXB, z_blk, 0)

    @pl.when(s == NS - 1)
    def _():
        pltpu.sync_copy(rows.at[pl.ds(0, NREM), :],
                        acc_sp.at[pl.ds(NBLK * CH, NREM), :])
        pltpu.sync_copy(expc.at[pl.ds(0, NREM), :],
                        denom_sp.at[pl.ds(NBLK * CH, NREM), :])
    plsc.subcore_barrier()

    def exp_rows(n):
        def per_edge(k, carry):
            ee = avs[k, :] + avd[k, :]
            ee = jnp.maximum(ee, 0.2 * ee)
            ex = jnp.exp(ee)
            expc[k, :] = jnp.where(head_mask, ex, 0.0)
            return carry
        lax.fori_loop(0, n, per_edge, 0)

    def load_edges(off, n, si, di, dn):
        pltpu.sync_copy(src_ref.at[pl.ds(ebase + off, n)], si)
        pltpu.sync_copy(dst_ref.at[pl.ds(ebase + off, n)], di)
        for l in range(n // L):
            dn[pl.ds(l * L, L)] = di[pl.ds(l * L, L)] + N
        pltpu.sync_copy(a_ref.at[si], avs.at[pl.ds(0, n), :])
        pltpu.sync_copy(a_ref.at[dn], avd.at[pl.ds(0, n), :])

    def p1_chunk(off, n, si, di, dn):
        load_edges(off, n, si, di, dn)
        exp_rows(n)
        pltpu.sync_copy(expc.at[pl.ds(0, n), :], denom_sp.at[di], add=True)

    def p1_loop(j, carry):
        p1_chunk(j * CH, CH, sidx, didx, dnidx)
        return carry
    lax.fori_loop(0, NFULL, p1_loop, 0)
    p1_chunk(NFULL * CH, REM, sidx16, didx16, dnidx16)
    plsc.subcore_barrier()

    coff = c * N

    def p2_chunk(off, n, si, di, wi, dn):
        load_edges(off, n, si, di, dn)
        for l in range(n // L):
            wi[pl.ds(l * L, L)] = si[pl.ds(l * L, L)] + coff
        pltpu.sync_copy(wx_ref.at[wi], rows.at[pl.ds(0, n), :])
        pltpu.sync_copy(denom_sp.at[di], dv.at[pl.ds(0, n), :])
        exp_rows(n)

        def per_edge(k, carry):
            q = expc[k, :] / (dv[k, :] + 2e-9)
            qbuf[...] = q
            q1 = q + plsc.load_gather(qbuf, [iota ^ 1])
            qbuf[...] = q1
            q2 = q1 + plsc.load_gather(qbuf, [iota ^ 2])
            plsc.store_scatter(coeff, [jnp.zeros((L,), jnp.int32) + k], q2,
                               mask=iota == 0)
            return carry
        lax.fori_loop(0, n, per_edge, 0)

        def scale(k, carry):
            cb = plsc.load_gather(coeff, [jnp.zeros((L,), jnp.int32) + k])
            for l in range(FH // L):
                rows[k, pl.ds(l * L, L)] = rows[k, pl.ds(l * L, L)] * cb
            return carry
        lax.fori_loop(0, n, scale, 0)
        pltpu.sync_copy(rows.at[pl.ds(0, n), :], acc_sp.at[di], add=True)

    def p2_loop(j, carry):
        p2_chunk(j * CH, CH, sidx, didx, wxidx, dnidx)
        return carry
    lax.fori_loop(0, NFULL, p2_loop, 0)
    p2_chunk(NFULL * CH, REM, sidx16, didx16, wxidx16, dnidx16)
    plsc.subcore_barrier()

    cfh = pl.multiple_of(c * FH, FH)

    def elu_rows(nrows):
        def rw(r, carry):
            for l in range(FH // L):
                v = outbuf[r, pl.ds(l * L, L)] * 0.25
                outbuf[r, pl.ds(l * L, L)] = jnp.where(
                    v > 0, v, jnp.exp(v) - 1.0)
            return carry
        lax.fori_loop(0, nrows, rw, 0)

    def p3_blk(k, carry):
        b = s + k * NS

        @pl.when(b < NBLK)
        def _():
            nb0 = pl.multiple_of(b * CH, CH)
            pltpu.sync_copy(acc_sp.at[pl.ds(nb0, CH), :], outbuf)
            elu_rows(CH)
            pltpu.sync_copy(outbuf, out_ref.at[pl.ds(nb0, CH), pl.ds(cfh, FH)])
        return carry
    lax.fori_loop(0, MAXB, p3_blk, 0)

    @pl.when(s == NS - 1)
    def _():
        nb0 = NBLK * CH
        pltpu.sync_copy(acc_sp.at[pl.ds(nb0, NREM), :],
                        outbuf.at[pl.ds(0, NREM), :])
        elu_rows(NREM)
        pltpu.sync_copy(outbuf.at[pl.ds(0, NREM), :],
                        out_ref.at[pl.ds(nb0, NREM), pl.ds(cfh, FH)])


@functools.partial(
    pl.kernel,
    out_type=jax.ShapeDtypeStruct((N, F), jnp.float32),
    mesh=plsc.VectorSubcoreMesh(core_axis_name="c", subcore_axis_name="s"),
    scratch_types=[
        pltpu.VMEM((CH,), jnp.int32),
        pltpu.VMEM((CH,), jnp.int32),
        pltpu.VMEM((CH,), jnp.int32),
        pltpu.VMEM((CH,), jnp.int32),
        pltpu.VMEM((REM,), jnp.int32),
        pltpu.VMEM((REM,), jnp.int32),
        pltpu.VMEM((REM,), jnp.int32),
        pltpu.VMEM((REM,), jnp.int32),
        pltpu.VMEM((CH, L), jnp.float32),
        pltpu.VMEM((CH, L), jnp.float32),
        pltpu.VMEM((CH, L), jnp.float32),
        pltpu.VMEM((CH, L), jnp.float32),
        pltpu.VMEM((CH, FH), jnp.float32),
        pltpu.VMEM((CH,), jnp.float32),
        pltpu.VMEM((L,), jnp.float32),
        pltpu.VMEM_SHARED((N, L), jnp.float32),
        pltpu.VMEM_SHARED((N, FH), jnp.float32),
    ],
    compiler_params=pltpu.CompilerParams(needs_layout_passes=False,
                                         use_tc_tiling_on_sc=False),
)
def _sc_edge_kernel(wx_ref, a_ref, src_ref, dst_ref, out_ref, *scratch):
    _sc_body(wx_ref, a_ref, src_ref, dst_ref, out_ref, *scratch)


def kernel(x, edge_index, W, att_W):
    src = edge_index[0].astype(jnp.int32)
    dst = edge_index[1].astype(jnp.int32)
    att32 = jnp.zeros((F, 2 * L), jnp.float32)
    att32 = att32.at[:, 0:4].set(att_W[:, :F].T)
    att32 = att32.at[:, L:L + 4].set(att_W[:, F:].T)
    wx2, a2 = _tc_matmul(x, W, att32)
    wx_cat = wx2.reshape(NC * N, FH)
    a_cat = a2.reshape(2 * N, L)
    return _sc_edge_kernel(wx_cat, a_cat, src, dst)


# speedup vs baseline: 14.8735x; 1.2447x over previous
import functools

import jax
import jax.numpy as jnp
from jax import lax
from jax.experimental import pallas as pl
from jax.experimental.pallas import tpu as pltpu
from jax.experimental.pallas import tpu_sc as plsc

N = 10000
E = 160000
F = 256
FH = 128
NC = 2
NS = 16
L = 16
EPT = E // NS
CH = 128
NFULL = EPT // CH
REM = EPT - NFULL * CH
NBLK = N // CH
NREM = N - NBLK * CH
MAXB = (NBLK + NS - 1) // NS


def _tc_body(x_ref, w_ref, att_ref, wx2_ref, a_ref):
    xb = x_ref[...]
    wx = lax.dot_general(xb, w_ref[...], (((1,), (1,)), ((), ())),
                         preferred_element_type=jnp.float32)
    wx2_ref[0] = wx[:, :FH]
    wx2_ref[1] = wx[:, FH:]
    ab = jnp.dot(wx, att_ref[...], preferred_element_type=jnp.float32)
    a_ref[0] = ab[:, :L]
    a_ref[1] = ab[:, L:]


def _tc_matmul(x, W, att16):
    nb = 1000
    grid = N // nb
    return pl.pallas_call(
        _tc_body,
        grid=(grid,),
        in_specs=[
            pl.BlockSpec((nb, F), lambda i: (i, 0)),
            pl.BlockSpec((F, F), lambda i: (0, 0)),
            pl.BlockSpec((F, 2 * L), lambda i: (0, 0)),
        ],
        out_specs=[
            pl.BlockSpec((NC, nb, FH), lambda i: (0, i, 0)),
            pl.BlockSpec((2, nb, L), lambda i: (0, i, 0)),
        ],
        out_shape=[
            jax.ShapeDtypeStruct((NC, N, FH), jnp.float32),
            jax.ShapeDtypeStruct((2, N, L), jnp.float32),
        ],
    )(x, W, att16)


def _sc_body(wx_ref, a_ref, src_ref, dst_ref, out_ref,
             sidx, didx, wxidx, dnidx, sidx16, didx16, wxidx16, dnidx16,
             avs, avd, expc, dv, rows, coeff, qf,
             denom_sp, acc_sp):
    outbuf = rows
    c = lax.axis_index("c")
    s = lax.axis_index("s")
    iota = lax.iota(jnp.int32, L)
    head_mask = iota < 4
    zero16 = jnp.zeros((L,), jnp.float32)
    ebase = pl.multiple_of(s * EPT, EPT)

    def zero_rows(k, carry):
        for l in range(FH // L):
            rows[k, pl.ds(l * L, L)] = zero16
        expc[k, :] = zero16
        return carry
    lax.fori_loop(0, CH, zero_rows, 0)

    def z_blk(k, carry):
        b = s + k * NS

        @pl.when(b < NBLK)
        def _():
            nb0 = pl.multiple_of(b * CH, CH)
            pltpu.sync_copy(rows, acc_sp.at[pl.ds(nb0, CH), :])
            pltpu.sync_copy(expc, denom_sp.at[pl.ds(nb0, CH), :])
        return carry
    lax.fori_loop(0, MAXB, z_blk, 0)

    @pl.when(s == NS - 1)
    def _():
        pltpu.sync_copy(rows.at[pl.ds(0, NREM), :],
                        acc_sp.at[pl.ds(NBLK * CH, NREM), :])
        pltpu.sync_copy(expc.at[pl.ds(0, NREM), :],
                        denom_sp.at[pl.ds(NBLK * CH, NREM), :])
    plsc.subcore_barrier()

    def exp_rows(n):
        def per_edge(k, carry):
            ee = avs[k, :] + avd[k, :]
            ee = jnp.maximum(ee, 0.2 * ee)
            ex = jnp.exp(ee)
            expc[k, :] = jnp.where(head_mask, ex, 0.0)
            return carry
        lax.fori_loop(0, n, per_edge, 0)

    def load_edges(off, n, si, di, dn):
        pltpu.sync_copy(src_ref.at[pl.ds(ebase + off, n)], si)
        pltpu.sync_copy(dst_ref.at[pl.ds(ebase + off, n)], di)
        for l in range(n // L):
            dn[pl.ds(l * L, L)] = di[pl.ds(l * L, L)] + N
        pltpu.sync_copy(a_ref.at[si], avs.at[pl.ds(0, n), :])
        pltpu.sync_copy(a_ref.at[dn], avd.at[pl.ds(0, n), :])

    def p1_chunk(off, n, si, di, dn):
        load_edges(off, n, si, di, dn)
        exp_rows(n)
        pltpu.sync_copy(expc.at[pl.ds(0, n), :], denom_sp.at[di], add=True)

    def p1_loop(j, carry):
        p1_chunk(j * CH, CH, sidx, didx, dnidx)
        return carry
    lax.fori_loop(0, NFULL, p1_loop, 0)
    p1_chunk(NFULL * CH, REM, sidx16, didx16, dnidx16)
    plsc.subcore_barrier()

    def rcp_rows(nrows):
        def rr(r, carry):
            dv[r, :] = 1.0 / (dv[r, :] + 2e-9)
            return carry
        lax.fori_loop(0, nrows, rr, 0)

    def r_blk(k, carry):
        b = s + k * NS

        @pl.when(b < NBLK)
        def _():
            nb0 = pl.multiple_of(b * CH, CH)
            pltpu.sync_copy(denom_sp.at[pl.ds(nb0, CH), :], dv)
            rcp_rows(CH)
            pltpu.sync_copy(dv, denom_sp.at[pl.ds(nb0, CH), :])
        return carry
    lax.fori_loop(0, MAXB, r_blk, 0)

    @pl.when(s == NS - 1)
    def _():
        pltpu.sync_copy(denom_sp.at[pl.ds(NBLK * CH, NREM), :],
                        dv.at[pl.ds(0, NREM), :])
        rcp_rows(NREM)
        pltpu.sync_copy(dv.at[pl.ds(0, NREM), :],
                        denom_sp.at[pl.ds(NBLK * CH, NREM), :])
    plsc.subcore_barrier()

    coff = c * N

    def p2_chunk(off, n, si, di, wi, dn):
        load_edges(off, n, si, di, dn)
        for l in range(n // L):
            wi[pl.ds(l * L, L)] = si[pl.ds(l * L, L)] + coff
        pltpu.sync_copy(wx_ref.at[wi], rows.at[pl.ds(0, n), :])
        pltpu.sync_copy(denom_sp.at[di], dv.at[pl.ds(0, n), :])

        def per_edge(k, carry):
            ee = avs[k, :] + avd[k, :]
            ee = jnp.maximum(ee, 0.2 * ee)
            qf[pl.ds(k * L, L)] = jnp.exp(ee) * dv[k, :]
            return carry
        lax.fori_loop(0, n, per_edge, 0)

        def cgrp(m, carry):
            i0 = (m * L + iota) * L
            v = (plsc.load_gather(qf, [i0])
                 + plsc.load_gather(qf, [i0 + 1])
                 + plsc.load_gather(qf, [i0 + 2])
                 + plsc.load_gather(qf, [i0 + 3]))
            coeff[pl.ds(m * L, L)] = v
            return carry
        lax.fori_loop(0, n // L, cgrp, 0)

        def scale(k, carry):
            cb = plsc.load_gather(coeff, [jnp.zeros((L,), jnp.int32) + k])
            for l in range(FH // L):
                rows[k, pl.ds(l * L, L)] = rows[k, pl.ds(l * L, L)] * cb
            return carry
        lax.fori_loop(0, n, scale, 0)
        pltpu.sync_copy(rows.at[pl.ds(0, n), :], acc_sp.at[di], add=True)

    def p2_loop(j, carry):
        p2_chunk(j * CH, CH, sidx, didx, wxidx, dnidx)
        return carry
    lax.fori_loop(0, NFULL, p2_loop, 0)
    p2_chunk(NFULL * CH, REM, sidx16, didx16, wxidx16, dnidx16)
    plsc.subcore_barrier()

    cfh = pl.multiple_of(c * FH, FH)

    def elu_rows(nrows):
        def rw(r, carry):
            for l in range(FH // L):
                v = outbuf[r, pl.ds(l * L, L)] * 0.25
                outbuf[r, pl.ds(l * L, L)] = jnp.where(
                    v > 0, v, jnp.exp(v) - 1.0)
            return carry
        lax.fori_loop(0, nrows, rw, 0)

    def p3_blk(k, carry):
        b = s + k * NS

        @pl.when(b < NBLK)
        def _():
            nb0 = pl.multiple_of(b * CH, CH)
            pltpu.sync_copy(acc_sp.at[pl.ds(nb0, CH), :], outbuf)
            elu_rows(CH)
            pltpu.sync_copy(outbuf, out_ref.at[pl.ds(nb0, CH), pl.ds(cfh, FH)])
        return carry
    lax.fori_loop(0, MAXB, p3_blk, 0)

    @pl.when(s == NS - 1)
    def _():
        nb0 = NBLK * CH
        pltpu.sync_copy(acc_sp.at[pl.ds(nb0, NREM), :],
                        outbuf.at[pl.ds(0, NREM), :])
        elu_rows(NREM)
        pltpu.sync_copy(outbuf.at[pl.ds(0, NREM), :],
                        out_ref.at[pl.ds(nb0, NREM), pl.ds(cfh, FH)])


@functools.partial(
    pl.kernel,
    out_type=jax.ShapeDtypeStruct((N, F), jnp.float32),
    mesh=plsc.VectorSubcoreMesh(core_axis_name="c", subcore_axis_name="s"),
    scratch_types=[
        pltpu.VMEM((CH,), jnp.int32),
        pltpu.VMEM((CH,), jnp.int32),
        pltpu.VMEM((CH,), jnp.int32),
        pltpu.VMEM((CH,), jnp.int32),
        pltpu.VMEM((REM,), jnp.int32),
        pltpu.VMEM((REM,), jnp.int32),
        pltpu.VMEM((REM,), jnp.int32),
        pltpu.VMEM((REM,), jnp.int32),
        pltpu.VMEM((CH, L), jnp.float32),
        pltpu.VMEM((CH, L), jnp.float32),
        pltpu.VMEM((CH, L), jnp.float32),
        pltpu.VMEM((CH, L), jnp.float32),
        pltpu.VMEM((CH, FH), jnp.float32),
        pltpu.VMEM((CH,), jnp.float32),
        pltpu.VMEM((CH * L,), jnp.float32),
        pltpu.VMEM_SHARED((N, L), jnp.float32),
        pltpu.VMEM_SHARED((N, FH), jnp.float32),
    ],
    compiler_params=pltpu.CompilerParams(needs_layout_passes=False,
                                         use_tc_tiling_on_sc=False),
)
def _sc_edge_kernel(wx_ref, a_ref, src_ref, dst_ref, out_ref, *scratch):
    _sc_body(wx_ref, a_ref, src_ref, dst_ref, out_ref, *scratch)


def kernel(x, edge_index, W, att_W):
    src = edge_index[0].astype(jnp.int32)
    dst = edge_index[1].astype(jnp.int32)
    att32 = jnp.zeros((F, 2 * L), jnp.float32)
    att32 = att32.at[:, 0:4].set(att_W[:, :F].T)
    att32 = att32.at[:, L:L + 4].set(att_W[:, F:].T)
    wx2, a2 = _tc_matmul(x, W, att32)
    wx_cat = wx2.reshape(NC * N, FH)
    a_cat = a2.reshape(2 * N, L)
    return _sc_edge_kernel(wx_cat, a_cat, src, dst)


# speedup vs baseline: 15.7321x; 1.0577x over previous
import functools

import jax
import jax.numpy as jnp
from jax import lax
from jax.experimental import pallas as pl
from jax.experimental.pallas import tpu as pltpu
from jax.experimental.pallas import tpu_sc as plsc

N = 10000
E = 160000
F = 256
FH = 128
NC = 2
NS = 16
L = 16
EPT = E // NS
CH = 80
NCH = EPT // CH
NBLK = N // CH
MAXB = (NBLK + NS - 1) // NS


def _tc_body(x_ref, w_ref, att_ref, wx2_ref, a_ref):
    xb = x_ref[...]
    wx = lax.dot_general(xb, w_ref[...], (((1,), (1,)), ((), ())),
                         preferred_element_type=jnp.float32)
    wx2_ref[0] = wx[:, :FH]
    wx2_ref[1] = wx[:, FH:]
    ab = jnp.dot(wx, att_ref[...], preferred_element_type=jnp.float32)
    a_ref[0] = ab[:, :L]
    a_ref[1] = ab[:, L:]


def _tc_matmul(x, W, att32):
    nb = 1000
    grid = N // nb
    return pl.pallas_call(
        _tc_body,
        grid=(grid,),
        in_specs=[
            pl.BlockSpec((nb, F), lambda i: (i, 0)),
            pl.BlockSpec((F, F), lambda i: (0, 0)),
            pl.BlockSpec((F, 2 * L), lambda i: (0, 0)),
        ],
        out_specs=[
            pl.BlockSpec((NC, nb, FH), lambda i: (0, i, 0)),
            pl.BlockSpec((2, nb, L), lambda i: (0, i, 0)),
        ],
        out_shape=[
            jax.ShapeDtypeStruct((NC, N, FH), jnp.float32),
            jax.ShapeDtypeStruct((2, N, L), jnp.float32),
        ],
    )(x, W, att32)


def _sc_body(wx_ref, a_ref, src_ref, dst_ref, out_ref,
             sidx, didx, wxidx, dnidx, avs, avd, dv, expc, rows, coeff, qf,
             semA, semB, semC, semD, denom_sp, acc_sp):
    c = lax.axis_index("c")
    s = lax.axis_index("s")
    iota = lax.iota(jnp.int32, L)
    head_mask = iota < 4
    zero16 = jnp.zeros((L,), jnp.float32)
    ebase = pl.multiple_of(s * EPT, 8)
    coff = c * N

    def idx_load(j, sl):
        off = ebase + j * CH
        pltpu.sync_copy(src_ref.at[pl.ds(off, CH)], sidx.at[sl])
        pltpu.sync_copy(dst_ref.at[pl.ds(off, CH)], didx.at[sl])

    def compute_dn(sl):
        for l in range(CH // L):
            dnidx[sl, pl.ds(l * L, L)] = didx[sl, pl.ds(l * L, L)] + N

    def compute_wi(sl):
        for l in range(CH // L):
            wxidx[sl, pl.ds(l * L, L)] = sidx[sl, pl.ds(l * L, L)] + coff

    def issue_p1(sl):
        return [
            pltpu.async_copy(a_ref.at[sidx.at[sl]], avs.at[sl], semA),
            pltpu.async_copy(a_ref.at[dnidx.at[sl]], avd.at[sl], semB),
        ]

    def issue_p2(sl):
        return issue_p1(sl) + [
            pltpu.async_copy(wx_ref.at[wxidx.at[sl]], rows.at[sl], semC),
            pltpu.async_copy(denom_sp.at[didx.at[sl]], dv.at[sl], semD),
        ]

    def zero_bufs(k, carry):
        for l in range(FH // L):
            rows[0, k, pl.ds(l * L, L)] = zero16
        expc[k, :] = zero16
        return carry
    lax.fori_loop(0, CH, zero_bufs, 0)

    def z_blk(k, carry):
        b = s + k * NS

        @pl.when(b < NBLK)
        def _():
            nb0 = pl.multiple_of(b * CH, 8)
            pltpu.sync_copy(rows.at[0], acc_sp.at[pl.ds(nb0, CH), :])
            pltpu.sync_copy(expc, denom_sp.at[pl.ds(nb0, CH), :])
        return carry
    lax.fori_loop(0, MAXB, z_blk, 0)
    plsc.subcore_barrier()

    idx_load(0, 0)
    compute_dn(0)
    for d in issue_p1(0):
        d.wait()

    def p1_body(j, carry):
        p = j & 1
        jn = jnp.minimum(j + 1, NCH - 1)
        idx_load(jn, 1 - p)
        compute_dn(1 - p)
        descs = issue_p1(1 - p)

        def per_edge(k, c2):
            ee = avs[p, k, :] + avd[p, k, :]
            ee = jnp.maximum(ee, 0.2 * ee)
            expc[k, :] = jnp.where(head_mask, jnp.exp(ee), 0.0)
            return c2
        lax.fori_loop(0, CH, per_edge, 0)
        pltpu.sync_copy(expc, denom_sp.at[didx.at[p]], add=True)
        for d in descs:
            d.wait()
        return carry
    lax.fori_loop(0, NCH, p1_body, 0)
    plsc.subcore_barrier()

    def r_blk(k, carry):
        b = s + k * NS

        @pl.when(b < NBLK)
        def _():
            nb0 = pl.multiple_of(b * CH, 8)
            pltpu.sync_copy(denom_sp.at[pl.ds(nb0, CH), :], expc)

            def rr(r, c2):
                expc[r, :] = 1.0 / (expc[r, :] + 2e-9)
                return c2
            lax.fori_loop(0, CH, rr, 0)
            pltpu.sync_copy(expc, denom_sp.at[pl.ds(nb0, CH), :])
        return carry
    lax.fori_loop(0, MAXB, r_blk, 0)
    plsc.subcore_barrier()

    idx_load(0, 0)
    compute_dn(0)
    compute_wi(0)
    for d in issue_p2(0):
        d.wait()

    def p2_body(j, carry):
        p = j & 1
        jn = jnp.minimum(j + 1, NCH - 1)
        idx_load(jn, 1 - p)
        compute_dn(1 - p)
        compute_wi(1 - p)
        descs = issue_p2(1 - p)

        def per_edge(k, c2):
            ee = avs[p, k, :] + avd[p, k, :]
            ee = jnp.maximum(ee, 0.2 * ee)
            qf[pl.ds(k * L, L)] = jnp.exp(ee) * dv[p, k, :]
            return c2
        lax.fori_loop(0, CH, per_edge, 0)

        def cgrp(m, c2):
            i0 = (m * L + iota) * L
            v = (plsc.load_gather(qf, [i0])
                 + plsc.load_gather(qf, [i0 + 1])
                 + plsc.load_gather(qf, [i0 + 2])
                 + plsc.load_gather(qf, [i0 + 3]))
            coeff[pl.ds(m * L, L)] = v
            return c2
        lax.fori_loop(0, CH // L, cgrp, 0)

        def scale(k, c2):
            cb = plsc.load_gather(coeff, [jnp.zeros((L,), jnp.int32) + k])
            for l in range(FH // L):
                rows[p, k, pl.ds(l * L, L)] = (
                    rows[p, k, pl.ds(l * L, L)] * cb)
            return c2
        lax.fori_loop(0, CH, scale, 0)
        pltpu.sync_copy(rows.at[p], acc_sp.at[didx.at[p]], add=True)
        for d in descs:
            d.wait()
        return carry
    lax.fori_loop(0, NCH, p2_body, 0)
    plsc.subcore_barrier()

    cfh = pl.multiple_of(c * FH, FH)

    def p3_blk(k, carry):
        b = s + k * NS

        @pl.when(b < NBLK)
        def _():
            nb0 = pl.multiple_of(b * CH, 8)
            pltpu.sync_copy(acc_sp.at[pl.ds(nb0, CH), :], rows.at[0])

            def rw(r, c2):
                for l in range(FH // L):
                    v = rows[0, r, pl.ds(l * L, L)] * 0.25
                    rows[0, r, pl.ds(l * L, L)] = jnp.where(
                        v > 0, v, jnp.exp(v) - 1.0)
                return c2
            lax.fori_loop(0, CH, rw, 0)
            pltpu.sync_copy(rows.at[0],
                            out_ref.at[pl.ds(nb0, CH), pl.ds(cfh, FH)])
        return carry
    lax.fori_loop(0, MAXB, p3_blk, 0)


@functools.partial(
    pl.kernel,
    out_type=jax.ShapeDtypeStruct((N, F), jnp.float32),
    mesh=plsc.VectorSubcoreMesh(core_axis_name="c", subcore_axis_name="s"),
    scratch_types=[
        pltpu.VMEM((2, CH), jnp.int32),
        pltpu.VMEM((2, CH), jnp.int32),
        pltpu.VMEM((2, CH), jnp.int32),
        pltpu.VMEM((2, CH), jnp.int32),
        pltpu.VMEM((2, CH, L), jnp.float32),
        pltpu.VMEM((2, CH, L), jnp.float32),
        pltpu.VMEM((2, CH, L), jnp.float32),
        pltpu.VMEM((CH, L), jnp.float32),
        pltpu.VMEM((2, CH, FH), jnp.float32),
        pltpu.VMEM((CH,), jnp.float32),
        pltpu.VMEM((CH * L,), jnp.float32),
        pltpu.SemaphoreType.DMA,
        pltpu.SemaphoreType.DMA,
        pltpu.SemaphoreType.DMA,
        pltpu.SemaphoreType.DMA,
        pltpu.VMEM_SHARED((N, L), jnp.float32),
        pltpu.VMEM_SHARED((N, FH), jnp.float32),
    ],
    compiler_params=pltpu.CompilerParams(needs_layout_passes=False,
                                         use_tc_tiling_on_sc=False),
)
def _sc_edge_kernel(wx_ref, a_ref, src_ref, dst_ref, out_ref, *scratch):
    _sc_body(wx_ref, a_ref, src_ref, dst_ref, out_ref, *scratch)


def kernel(x, edge_index, W, att_W):
    src = edge_index[0].astype(jnp.int32)
    dst = edge_index[1].astype(jnp.int32)
    att32 = jnp.zeros((F, 2 * L), jnp.float32)
    att32 = att32.at[:, 0:4].set(att_W[:, :F].T)
    att32 = att32.at[:, L:L + 4].set(att_W[:, F:].T)
    wx2, a2 = _tc_matmul(x, W, att32)
    wx_cat = wx2.reshape(NC * N, FH)
    a_cat = a2.reshape(2 * N, L)
    return _sc_edge_kernel(wx_cat, a_cat, src, dst)


# speedup vs baseline: 22.8648x; 1.4534x over previous
import functools

import jax
import jax.numpy as jnp
from jax import lax
from jax.experimental import pallas as pl
from jax.experimental.pallas import tpu as pltpu
from jax.experimental.pallas import tpu_sc as plsc

N = 10000
E = 160000
F = 256
FH = 128
NC = 2
NS = 16
L = 16
EPT = E // NS
CH = 80
NCH = EPT // CH
NBLK = N // CH
MAXB = (NBLK + NS - 1) // NS


def _tc_body(x_ref, w_ref, att_ref, wx2_ref, a_ref):
    xb = x_ref[...]
    wx = lax.dot_general(xb, w_ref[...], (((1,), (1,)), ((), ())),
                         preferred_element_type=jnp.float32)
    wx2_ref[0] = wx[:, :FH]
    wx2_ref[1] = wx[:, FH:]
    ab = jnp.dot(wx, att_ref[...], preferred_element_type=jnp.float32)
    a_ref[0] = ab[:, :L]
    a_ref[1] = ab[:, L:]


def _tc_matmul(x, W, att32):
    nb = 1000
    grid = N // nb
    return pl.pallas_call(
        _tc_body,
        grid=(grid,),
        in_specs=[
            pl.BlockSpec((nb, F), lambda i: (i, 0)),
            pl.BlockSpec((F, F), lambda i: (0, 0)),
            pl.BlockSpec((F, 2 * L), lambda i: (0, 0)),
        ],
        out_specs=[
            pl.BlockSpec((NC, nb, FH), lambda i: (0, i, 0)),
            pl.BlockSpec((2, nb, L), lambda i: (0, i, 0)),
        ],
        out_shape=[
            jax.ShapeDtypeStruct((NC, N, FH), jnp.float32),
            jax.ShapeDtypeStruct((2, N, L), jnp.float32),
        ],
    )(x, W, att32)


def _sc_body(wx_ref, a_ref, src_ref, dst_ref, out_ref,
             sidx, didx, wxidx, dnidx, avs, avd, dv, expc, rows, coeff, qf,
             semA, semB, semC, semD, denom_sp, acc_sp):
    c = lax.axis_index("c")
    s = lax.axis_index("s")
    iota = lax.iota(jnp.int32, L)
    head_mask = iota < 4
    zero16 = jnp.zeros((L,), jnp.float32)
    ebase = pl.multiple_of(s * EPT, 8)
    coff = c * N

    def idx_load(j, sl):
        off = ebase + j * CH
        pltpu.sync_copy(src_ref.at[pl.ds(off, CH)], sidx.at[sl])
        pltpu.sync_copy(dst_ref.at[pl.ds(off, CH)], didx.at[sl])

    def compute_dn(sl):
        for l in range(CH // L):
            dnidx[sl, pl.ds(l * L, L)] = didx[sl, pl.ds(l * L, L)] + N

    def compute_wi(sl):
        for l in range(CH // L):
            wxidx[sl, pl.ds(l * L, L)] = sidx[sl, pl.ds(l * L, L)] + coff

    def issue_p1(sl):
        return [
            pltpu.async_copy(a_ref.at[sidx.at[sl]], avs.at[sl], semA),
            pltpu.async_copy(a_ref.at[dnidx.at[sl]], avd.at[sl], semB),
        ]

    def issue_p2(sl):
        return issue_p1(sl) + [
            pltpu.async_copy(wx_ref.at[wxidx.at[sl]], rows.at[sl], semC),
            pltpu.async_copy(denom_sp.at[didx.at[sl]], dv.at[sl], semD),
        ]

    def zero_bufs(k, carry):
        for l in range(FH // L):
            rows[0, k, pl.ds(l * L, L)] = zero16
        expc[k, :] = zero16
        return carry
    lax.fori_loop(0, CH, zero_bufs, 0)

    def z_blk(k, carry):
        b = s + k * NS

        @pl.when(b < NBLK)
        def _():
            nb0 = pl.multiple_of(b * CH, 8)
            pltpu.sync_copy(rows.at[0], acc_sp.at[pl.ds(nb0, CH), :])
            pltpu.sync_copy(expc, denom_sp.at[pl.ds(nb0, CH), :])
        return carry
    lax.fori_loop(0, MAXB, z_blk, 0)
    plsc.subcore_barrier()

    idx_load(0, 0)
    compute_dn(0)
    for d in issue_p1(0):
        d.wait()

    def p1_body(j, carry):
        p = j & 1
        jn = jnp.minimum(j + 1, NCH - 1)
        idx_load(jn, 1 - p)
        compute_dn(1 - p)
        descs = issue_p1(1 - p)

        @plsc.parallel_loop(0, CH, unroll=4)
        def _(k):
            ee = avs[p, k, :] + avd[p, k, :]
            ee = jnp.maximum(ee, 0.2 * ee)
            expc[k, :] = jnp.where(head_mask, jnp.exp(ee), 0.0)
        pltpu.sync_copy(expc, denom_sp.at[didx.at[p]], add=True)
        for d in descs:
            d.wait()
        return carry
    lax.fori_loop(0, NCH, p1_body, 0)
    plsc.subcore_barrier()

    def r_blk(k, carry):
        b = s + k * NS

        @pl.when(b < NBLK)
        def _():
            nb0 = pl.multiple_of(b * CH, 8)
            pltpu.sync_copy(denom_sp.at[pl.ds(nb0, CH), :], expc)

            def rr(r, c2):
                expc[r, :] = 1.0 / (expc[r, :] + 2e-9)
                return c2
            lax.fori_loop(0, CH, rr, 0)
            pltpu.sync_copy(expc, denom_sp.at[pl.ds(nb0, CH), :])
        return carry
    lax.fori_loop(0, MAXB, r_blk, 0)
    plsc.subcore_barrier()

    idx_load(0, 0)
    compute_dn(0)
    compute_wi(0)
    for d in issue_p2(0):
        d.wait()

    def p2_body(j, carry):
        p = j & 1
        jn = jnp.minimum(j + 1, NCH - 1)
        idx_load(jn, 1 - p)
        compute_dn(1 - p)
        compute_wi(1 - p)
        descs = issue_p2(1 - p)

        @plsc.parallel_loop(0, CH, unroll=4)
        def _(k):
            ee = avs[p, k, :] + avd[p, k, :]
            ee = jnp.maximum(ee, 0.2 * ee)
            qf[pl.ds(k * L, L)] = jnp.exp(ee) * dv[p, k, :]

        @plsc.parallel_loop(0, CH // L, unroll=5)
        def _(m):
            i0 = (m * L + iota) * L
            v = (plsc.load_gather(qf, [i0])
                 + plsc.load_gather(qf, [i0 + 1])
                 + plsc.load_gather(qf, [i0 + 2])
                 + plsc.load_gather(qf, [i0 + 3]))
            coeff[pl.ds(m * L, L)] = v

        @plsc.parallel_loop(0, CH, unroll=2)
        def _(k):
            cb = plsc.load_gather(coeff, [jnp.zeros((L,), jnp.int32) + k])
            for l in range(FH // L):
                rows[p, k, pl.ds(l * L, L)] = (
                    rows[p, k, pl.ds(l * L, L)] * cb)
        pltpu.sync_copy(rows.at[p], acc_sp.at[didx.at[p]], add=True)
        for d in descs:
            d.wait()
        return carry
    lax.fori_loop(0, NCH, p2_body, 0)
    plsc.subcore_barrier()

    cfh = pl.multiple_of(c * FH, FH)

    def p3_blk(k, carry):
        b = s + k * NS

        @pl.when(b < NBLK)
        def _():
            nb0 = pl.multiple_of(b * CH, 8)
            pltpu.sync_copy(acc_sp.at[pl.ds(nb0, CH), :], rows.at[0])

            def rw(r, c2):
                for l in range(FH // L):
                    v = rows[0, r, pl.ds(l * L, L)] * 0.25
                    rows[0, r, pl.ds(l * L, L)] = jnp.where(
                        v > 0, v, jnp.exp(v) - 1.0)
                return c2
            lax.fori_loop(0, CH, rw, 0)
            pltpu.sync_copy(rows.at[0],
                            out_ref.at[pl.ds(nb0, CH), pl.ds(cfh, FH)])
        return carry
    lax.fori_loop(0, MAXB, p3_blk, 0)


@functools.partial(
    pl.kernel,
    out_type=jax.ShapeDtypeStruct((N, F), jnp.float32),
    mesh=plsc.VectorSubcoreMesh(core_axis_name="c", subcore_axis_name="s"),
    scratch_types=[
        pltpu.VMEM((2, CH), jnp.int32),
        pltpu.VMEM((2, CH), jnp.int32),
        pltpu.VMEM((2, CH), jnp.int32),
        pltpu.VMEM((2, CH), jnp.int32),
        pltpu.VMEM((2, CH, L), jnp.float32),
        pltpu.VMEM((2, CH, L), jnp.float32),
        pltpu.VMEM((2, CH, L), jnp.float32),
        pltpu.VMEM((CH, L), jnp.float32),
        pltpu.VMEM((2, CH, FH), jnp.float32),
        pltpu.VMEM((CH,), jnp.float32),
        pltpu.VMEM((CH * L,), jnp.float32),
        pltpu.SemaphoreType.DMA,
        pltpu.SemaphoreType.DMA,
        pltpu.SemaphoreType.DMA,
        pltpu.SemaphoreType.DMA,
        pltpu.VMEM_SHARED((N, L), jnp.float32),
        pltpu.VMEM_SHARED((N, FH), jnp.float32),
    ],
    compiler_params=pltpu.CompilerParams(needs_layout_passes=False,
                                         use_tc_tiling_on_sc=False),
)
def _sc_edge_kernel(wx_ref, a_ref, src_ref, dst_ref, out_ref, *scratch):
    _sc_body(wx_ref, a_ref, src_ref, dst_ref, out_ref, *scratch)


def kernel(x, edge_index, W, att_W):
    src = edge_index[0].astype(jnp.int32)
    dst = edge_index[1].astype(jnp.int32)
    att32 = jnp.zeros((F, 2 * L), jnp.float32)
    att32 = att32.at[:, 0:4].set(att_W[:, :F].T)
    att32 = att32.at[:, L:L + 4].set(att_W[:, F:].T)
    wx2, a2 = _tc_matmul(x, W, att32)
    wx_cat = wx2.reshape(NC * N, FH)
    a_cat = a2.reshape(2 * N, L)
    return _sc_edge_kernel(wx_cat, a_cat, src, dst)


# speedup vs baseline: 28.4002x; 1.2421x over previous
import functools

import jax
import jax.numpy as jnp
from jax import lax
from jax.experimental import pallas as pl
from jax.experimental.pallas import tpu as pltpu
from jax.experimental.pallas import tpu_sc as plsc

N = 10000
E = 160000
F = 256
FH = 128
NC = 2
NS = 16
L = 16
EPT = E // NS
CH = 80
NCH = EPT // CH
NBLK = N // CH
MAXB = (NBLK + NS - 1) // NS


def _tc_body(x_ref, w_ref, att_ref, wx2_ref, a_ref):
    xb = x_ref[...]
    wx = lax.dot_general(xb, w_ref[...], (((1,), (1,)), ((), ())),
                         preferred_element_type=jnp.float32)
    wx2_ref[0] = wx[:, :FH]
    wx2_ref[1] = wx[:, FH:]
    ab = jnp.dot(wx, att_ref[...], preferred_element_type=jnp.float32)
    a_ref[0] = ab[:, :L]
    a_ref[1] = ab[:, L:]


def _tc_matmul(x, W, att32):
    nb = 1000
    grid = N // nb
    return pl.pallas_call(
        _tc_body,
        grid=(grid,),
        in_specs=[
            pl.BlockSpec((nb, F), lambda i: (i, 0)),
            pl.BlockSpec((F, F), lambda i: (0, 0)),
            pl.BlockSpec((F, 2 * L), lambda i: (0, 0)),
        ],
        out_specs=[
            pl.BlockSpec((NC, nb, FH), lambda i: (0, i, 0)),
            pl.BlockSpec((2, nb, L), lambda i: (0, i, 0)),
        ],
        out_shape=[
            jax.ShapeDtypeStruct((NC, N, FH), jnp.float32),
            jax.ShapeDtypeStruct((2, N, L), jnp.float32),
        ],
    )(x, W, att32)


def _sc_body(wx_ref, a_ref, eidx_ref, out_ref,
             sedge, wxidx, dnidx, avs, avd, dv, expc, rows, coeff, qf,
             semA, semB, semC, semD, denom_sp, acc_sp):
    c = lax.axis_index("c")
    s = lax.axis_index("s")
    iota = lax.iota(jnp.int32, L)
    head_mask = iota < 4
    zero16 = jnp.zeros((L,), jnp.float32)
    ebase = pl.multiple_of(s * EPT, 8)
    coff = c * N

    def idx_load(j, sl):
        off = ebase + j * CH
        pltpu.sync_copy(eidx_ref.at[:, pl.ds(off, CH)], sedge.at[sl])

    def compute_dn(sl):
        for l in range(CH // L):
            dnidx[sl, pl.ds(l * L, L)] = sedge[sl, 1, pl.ds(l * L, L)] + N

    def compute_wi(sl):
        for l in range(CH // L):
            wxidx[sl, pl.ds(l * L, L)] = sedge[sl, 0, pl.ds(l * L, L)] + coff

    def issue_p1(sl):
        return [
            pltpu.async_copy(a_ref.at[sedge.at[sl, 0]], avs.at[sl], semA),
            pltpu.async_copy(a_ref.at[dnidx.at[sl]], avd.at[sl], semB),
        ]

    def issue_p2(sl):
        return issue_p1(sl) + [
            pltpu.async_copy(wx_ref.at[wxidx.at[sl]], rows.at[sl], semC),
            pltpu.async_copy(denom_sp.at[sedge.at[sl, 1]], dv.at[sl], semD),
        ]

    def zero_bufs(k, carry):
        for l in range(FH // L):
            rows[0, k, pl.ds(l * L, L)] = zero16
        expc[k, :] = zero16
        return carry
    lax.fori_loop(0, CH, zero_bufs, 0)

    def z_blk(k, carry):
        b = s + k * NS

        @pl.when(b < NBLK)
        def _():
            nb0 = pl.multiple_of(b * CH, 8)
            pltpu.sync_copy(rows.at[0], acc_sp.at[pl.ds(nb0, CH), :])
            pltpu.sync_copy(expc, denom_sp.at[pl.ds(nb0, CH), :])
        return carry
    lax.fori_loop(0, MAXB, z_blk, 0)
    plsc.subcore_barrier()

    idx_load(0, 0)
    compute_dn(0)
    for d in issue_p1(0):
        d.wait()

    def p1_body(j, carry):
        p = j & 1
        jn = jnp.minimum(j + 1, NCH - 1)
        idx_load(jn, 1 - p)
        compute_dn(1 - p)
        descs = issue_p1(1 - p)

        @plsc.parallel_loop(0, CH, unroll=4)
        def _(k):
            ee = avs[p, k, :] + avd[p, k, :]
            ee = jnp.maximum(ee, 0.2 * ee)
            expc[k, :] = jnp.where(head_mask, jnp.exp(ee), 0.0)
        pltpu.sync_copy(expc, denom_sp.at[sedge.at[p, 1]], add=True)
        for d in descs:
            d.wait()
        return carry
    lax.fori_loop(0, NCH, p1_body, 0)
    plsc.subcore_barrier()

    def r_blk(k, carry):
        b = s + k * NS

        @pl.when(b < NBLK)
        def _():
            nb0 = pl.multiple_of(b * CH, 8)
            pltpu.sync_copy(denom_sp.at[pl.ds(nb0, CH), :], expc)

            @plsc.parallel_loop(0, CH, unroll=4)
            def _(r):
                expc[r, :] = 1.0 / (expc[r, :] + 2e-9)
            pltpu.sync_copy(expc, denom_sp.at[pl.ds(nb0, CH), :])
        return carry
    lax.fori_loop(0, MAXB, r_blk, 0)
    plsc.subcore_barrier()

    idx_load(0, 0)
    compute_dn(0)
    compute_wi(0)
    for d in issue_p2(0):
        d.wait()

    def p2_body(j, carry):
        p = j & 1
        jn = jnp.minimum(j + 1, NCH - 1)
        idx_load(jn, 1 - p)
        compute_dn(1 - p)
        compute_wi(1 - p)
        descs = issue_p2(1 - p)

        @plsc.parallel_loop(0, CH, unroll=4)
        def _(k):
            ee = avs[p, k, :] + avd[p, k, :]
            ee = jnp.maximum(ee, 0.2 * ee)
            qf[pl.ds(k * L, L)] = jnp.exp(ee) * dv[p, k, :]

        @plsc.parallel_loop(0, CH // L, unroll=5)
        def _(m):
            i0 = (m * L + iota) * L
            v = (plsc.load_gather(qf, [i0])
                 + plsc.load_gather(qf, [i0 + 1])
                 + plsc.load_gather(qf, [i0 + 2])
                 + plsc.load_gather(qf, [i0 + 3]))
            coeff[pl.ds(m * L, L)] = v

        @plsc.parallel_loop(0, CH, unroll=4)
        def _(k):
            cb = plsc.load_gather(coeff, [jnp.zeros((L,), jnp.int32) + k])
            for l in range(FH // L):
                rows[p, k, pl.ds(l * L, L)] = (
                    rows[p, k, pl.ds(l * L, L)] * cb)
        pltpu.sync_copy(rows.at[p], acc_sp.at[sedge.at[p, 1]], add=True)
        for d in descs:
            d.wait()
        return carry
    lax.fori_loop(0, NCH, p2_body, 0)
    plsc.subcore_barrier()

    cfh = pl.multiple_of(c * FH, FH)

    def p3_blk(k, carry):
        b = s + k * NS

        @pl.when(b < NBLK)
        def _():
            nb0 = pl.multiple_of(b * CH, 8)
            pltpu.sync_copy(acc_sp.at[pl.ds(nb0, CH), :], rows.at[0])

            @plsc.parallel_loop(0, CH, unroll=2)
            def _(r):
                for l in range(FH // L):
                    v = rows[0, r, pl.ds(l * L, L)] * 0.25
                    rows[0, r, pl.ds(l * L, L)] = jnp.where(
                        v > 0, v, jnp.exp(v) - 1.0)
            pltpu.sync_copy(rows.at[0],
                            out_ref.at[pl.ds(nb0, CH), pl.ds(cfh, FH)])
        return carry
    lax.fori_loop(0, MAXB, p3_blk, 0)


@functools.partial(
    pl.kernel,
    out_type=jax.ShapeDtypeStruct((N, F), jnp.float32),
    mesh=plsc.VectorSubcoreMesh(core_axis_name="c", subcore_axis_name="s"),
    scratch_types=[
        pltpu.VMEM((2, 2, CH), jnp.int32),
        pltpu.VMEM((2, CH), jnp.int32),
        pltpu.VMEM((2, CH), jnp.int32),
        pltpu.VMEM((2, CH, L), jnp.float32),
        pltpu.VMEM((2, CH, L), jnp.float32),
        pltpu.VMEM((2, CH, L), jnp.float32),
        pltpu.VMEM((CH, L), jnp.float32),
        pltpu.VMEM((2, CH, FH), jnp.float32),
        pltpu.VMEM((CH,), jnp.float32),
        pltpu.VMEM((CH * L,), jnp.float32),
        pltpu.SemaphoreType.DMA,
        pltpu.SemaphoreType.DMA,
        pltpu.SemaphoreType.DMA,
        pltpu.SemaphoreType.DMA,
        pltpu.VMEM_SHARED((N, L), jnp.float32),
        pltpu.VMEM_SHARED((N, FH), jnp.float32),
    ],
    compiler_params=pltpu.CompilerParams(needs_layout_passes=False,
                                         use_tc_tiling_on_sc=False),
)
def _sc_edge_kernel(wx_ref, a_ref, eidx_ref, out_ref, *scratch):
    _sc_body(wx_ref, a_ref, eidx_ref, out_ref, *scratch)


def kernel(x, edge_index, W, att_W):
    eidx = edge_index.astype(jnp.int32)
    att32 = jnp.zeros((F, 2 * L), jnp.float32)
    att32 = att32.at[:, 0:4].set(att_W[:, :F].T)
    att32 = att32.at[:, L:L + 4].set(att_W[:, F:].T)
    wx2, a2 = _tc_matmul(x, W, att32)
    wx_cat = wx2.reshape(NC * N, FH)
    a_cat = a2.reshape(2 * N, L)
    return _sc_edge_kernel(wx_cat, a_cat, eidx)


# speedup vs baseline: 33.4783x; 1.1788x over previous
import functools

import jax
import jax.numpy as jnp
from jax import lax
from jax.experimental import pallas as pl
from jax.experimental.pallas import tpu as pltpu
from jax.experimental.pallas import tpu_sc as plsc

N = 10000
E = 160000
F = 256
FH = 128
NC = 2
NS = 16
L = 16
EPT = E // NS
CH = 80
NCH = EPT // CH
NBLK = N // CH
MAXB = (NBLK + NS - 1) // NS


def _tc_body(x_ref, w_ref, att_ref, wx2_ref, a_ref):
    xb = x_ref[...]
    wx = lax.dot_general(xb, w_ref[...], (((1,), (1,)), ((), ())),
                         preferred_element_type=jnp.float32)
    wx2_ref[0] = wx[:, :FH]
    wx2_ref[1] = wx[:, FH:]
    ab = jnp.dot(wx, att_ref[...], preferred_element_type=jnp.float32)
    a_ref[0] = ab[:, :L]
    a_ref[1] = ab[:, L:]


def _tc_matmul(x, W, att32):
    nb = 1000
    grid = N // nb
    return pl.pallas_call(
        _tc_body,
        grid=(grid,),
        in_specs=[
            pl.BlockSpec((nb, F), lambda i: (i, 0)),
            pl.BlockSpec((F, F), lambda i: (0, 0)),
            pl.BlockSpec((F, 2 * L), lambda i: (0, 0)),
        ],
        out_specs=[
            pl.BlockSpec((NC, nb, FH), lambda i: (0, i, 0)),
            pl.BlockSpec((2, nb, L), lambda i: (0, i, 0)),
        ],
        out_shape=[
            jax.ShapeDtypeStruct((NC, N, FH), jnp.float32),
            jax.ShapeDtypeStruct((2, N, L), jnp.float32),
        ],
    )(x, W, att32)


def _sc_body(wx_ref, a_ref, eidx_ref, out_ref,
             sedge, wxidx, dnidx, avs, avd, dv, expc, rows, coeff, qf,
             semA, semB, semC, semD, semE, denom_sp, acc_sp):
    c = lax.axis_index("c")
    s = lax.axis_index("s")
    iota = lax.iota(jnp.int32, L)
    head_mask = iota < 4
    zero16 = jnp.zeros((L,), jnp.float32)
    ebase = pl.multiple_of(s * EPT, 8)
    coff = c * N

    def idx_load(j, sl):
        off = ebase + j * CH
        pltpu.sync_copy(eidx_ref.at[:, pl.ds(off, CH)], sedge.at[sl])

    def idx_issue(j, sl):
        off = ebase + j * CH
        pltpu.async_copy(eidx_ref.at[:, pl.ds(off, CH)], sedge.at[sl], semE)

    def idx_wait(j, sl):
        off = ebase + j * CH
        pltpu.make_async_copy(eidx_ref.at[:, pl.ds(off, CH)], sedge.at[sl],
                              semE).wait()

    def compute_dn(sl):
        for l in range(CH // L):
            dnidx[sl, pl.ds(l * L, L)] = sedge[sl, 1, pl.ds(l * L, L)] + N

    def compute_wi(sl):
        for l in range(CH // L):
            wxidx[sl, pl.ds(l * L, L)] = sedge[sl, 0, pl.ds(l * L, L)] + coff

    def issue_p1(sl):
        return [
            pltpu.async_copy(a_ref.at[sedge.at[sl, 0]], avs.at[sl], semA),
            pltpu.async_copy(a_ref.at[dnidx.at[sl]], avd.at[sl], semB),
        ]

    def issue_p2(sl):
        return issue_p1(sl) + [
            pltpu.async_copy(wx_ref.at[wxidx.at[sl]], rows.at[sl], semC),
            pltpu.async_copy(denom_sp.at[sedge.at[sl, 1]], dv.at[sl], semD),
        ]

    def zero_bufs(k, carry):
        for l in range(FH // L):
            rows[0, k, pl.ds(l * L, L)] = zero16
        expc[k, :] = zero16
        return carry
    lax.fori_loop(0, CH, zero_bufs, 0)

    def z_blk(k, carry):
        b = s + k * NS

        @pl.when(b < NBLK)
        def _():
            nb0 = pl.multiple_of(b * CH, 8)
            pltpu.sync_copy(rows.at[0], acc_sp.at[pl.ds(nb0, CH), :])
            pltpu.sync_copy(expc, denom_sp.at[pl.ds(nb0, CH), :])
        return carry
    lax.fori_loop(0, MAXB, z_blk, 0)
    plsc.subcore_barrier()

    idx_load(0, 0)
    compute_dn(0)
    for d in issue_p1(0):
        d.wait()
    idx_issue(1, 1)

    def p1_body(j, carry):
        p = j & 1
        jn = jnp.minimum(j + 1, NCH - 1)
        jn2 = jnp.minimum(j + 2, NCH - 1)
        idx_wait(jn, 1 - p)
        compute_dn(1 - p)
        descs = issue_p1(1 - p)

        @plsc.parallel_loop(0, CH, unroll=4)
        def _(k):
            ee = avs[p, k, :] + avd[p, k, :]
            ee = jnp.maximum(ee, 0.2 * ee)
            expc[k, :] = jnp.where(head_mask, jnp.exp(ee), 0.0)
        pltpu.sync_copy(expc, denom_sp.at[sedge.at[p, 1]], add=True)
        idx_issue(jn2, p)
        for d in descs:
            d.wait()
        return carry
    lax.fori_loop(0, NCH, p1_body, 0)
    idx_wait(NCH - 1, (NCH - 1) & 1)
    plsc.subcore_barrier()

    def r_blk(k, carry):
        b = s + k * NS

        @pl.when(b < NBLK)
        def _():
            nb0 = pl.multiple_of(b * CH, 8)
            pltpu.sync_copy(denom_sp.at[pl.ds(nb0, CH), :], expc)

            @plsc.parallel_loop(0, CH, unroll=4)
            def _(r):
                expc[r, :] = 1.0 / (expc[r, :] + 2e-9)
            pltpu.sync_copy(expc, denom_sp.at[pl.ds(nb0, CH), :])
        return carry
    lax.fori_loop(0, MAXB, r_blk, 0)
    plsc.subcore_barrier()

    idx_load(0, 0)
    compute_dn(0)
    compute_wi(0)
    for d in issue_p2(0):
        d.wait()
    idx_issue(1, 1)

    def p2_body(j, carry):
        p = j & 1
        jn = jnp.minimum(j + 1, NCH - 1)
        jn2 = jnp.minimum(j + 2, NCH - 1)
        idx_wait(jn, 1 - p)
        compute_dn(1 - p)
        compute_wi(1 - p)
        descs = issue_p2(1 - p)

        @plsc.parallel_loop(0, CH, unroll=4)
        def _(k):
            ee = avs[p, k, :] + avd[p, k, :]
            ee = jnp.maximum(ee, 0.2 * ee)
            qf[pl.ds(k * L, L)] = jnp.exp(ee) * dv[p, k, :]

        @plsc.parallel_loop(0, CH // L, unroll=5)
        def _(m):
            i0 = (m * L + iota) * L
            v = (plsc.load_gather(qf, [i0])
                 + plsc.load_gather(qf, [i0 + 1])
                 + plsc.load_gather(qf, [i0 + 2])
                 + plsc.load_gather(qf, [i0 + 3]))
            coeff[pl.ds(m * L, L)] = v

        @plsc.parallel_loop(0, CH, unroll=4)
        def _(k):
            cb = plsc.load_gather(coeff, [jnp.zeros((L,), jnp.int32) + k])
            for l in range(FH // L):
                rows[p, k, pl.ds(l * L, L)] = (
                    rows[p, k, pl.ds(l * L, L)] * cb)
        pltpu.sync_copy(rows.at[p], acc_sp.at[sedge.at[p, 1]], add=True)
        idx_issue(jn2, p)
        for d in descs:
            d.wait()
        return carry
    lax.fori_loop(0, NCH, p2_body, 0)
    idx_wait(NCH - 1, (NCH - 1) & 1)
    plsc.subcore_barrier()

    cfh = pl.multiple_of(c * FH, FH)

    def p3_blk(k, carry):
        b = s + k * NS

        @pl.when(b < NBLK)
        def _():
            nb0 = pl.multiple_of(b * CH, 8)
            pltpu.sync_copy(acc_sp.at[pl.ds(nb0, CH), :], rows.at[0])

            @plsc.parallel_loop(0, CH, unroll=2)
            def _(r):
                for l in range(FH // L):
                    v = rows[0, r, pl.ds(l * L, L)] * 0.25
                    rows[0, r, pl.ds(l * L, L)] = jnp.where(
                        v > 0, v, jnp.exp(v) - 1.0)
            pltpu.sync_copy(rows.at[0],
                            out_ref.at[pl.ds(nb0, CH), pl.ds(cfh, FH)])
        return carry
    lax.fori_loop(0, MAXB, p3_blk, 0)


@functools.partial(
    pl.kernel,
    out_type=jax.ShapeDtypeStruct((N, F), jnp.float32),
    mesh=plsc.VectorSubcoreMesh(core_axis_name="c", subcore_axis_name="s"),
    scratch_types=[
        pltpu.VMEM((2, 2, CH), jnp.int32),
        pltpu.VMEM((2, CH), jnp.int32),
        pltpu.VMEM((2, CH), jnp.int32),
        pltpu.VMEM((2, CH, L), jnp.float32),
        pltpu.VMEM((2, CH, L), jnp.float32),
        pltpu.VMEM((2, CH, L), jnp.float32),
        pltpu.VMEM((CH, L), jnp.float32),
        pltpu.VMEM((2, CH, FH), jnp.float32),
        pltpu.VMEM((CH,), jnp.float32),
        pltpu.VMEM((CH * L,), jnp.float32),
        pltpu.SemaphoreType.DMA,
        pltpu.SemaphoreType.DMA,
        pltpu.SemaphoreType.DMA,
        pltpu.SemaphoreType.DMA,
        pltpu.SemaphoreType.DMA,
        pltpu.VMEM_SHARED((N, L), jnp.float32),
        pltpu.VMEM_SHARED((N, FH), jnp.float32),
    ],
    compiler_params=pltpu.CompilerParams(needs_layout_passes=False,
                                         use_tc_tiling_on_sc=False),
)
def _sc_edge_kernel(wx_ref, a_ref, eidx_ref, out_ref, *scratch):
    _sc_body(wx_ref, a_ref, eidx_ref, out_ref, *scratch)


def kernel(x, edge_index, W, att_W):
    eidx = edge_index.astype(jnp.int32)
    att32 = jnp.zeros((F, 2 * L), jnp.float32)
    att32 = att32.at[:, 0:4].set(att_W[:, :F].T)
    att32 = att32.at[:, L:L + 4].set(att_W[:, F:].T)
    wx2, a2 = _tc_matmul(x, W, att32)
    wx_cat = wx2.reshape(NC * N, FH)
    a_cat = a2.reshape(2 * N, L)
    return _sc_edge_kernel(wx_cat, a_cat, eidx)
